# baseline (speedup 1.0000x reference)
import functools

import jax
import jax.numpy as jnp
import numpy as np
from jax import lax
from jax.experimental import pallas as pl
from jax.experimental.pallas import tpu as pltpu
from jax.experimental.pallas import tpu_sc as plsc

_IGNORE_THRESH = 0.7
_MIN_KEPT = 100000
_SB_WEIGHTS = 0.5

_NC = 2
_NS = 16
_NW = _NC * _NS
_LANES = 16
_B_SC = 2

_LN2 = np.float32(0.6931471805599453)
_LOG2_POLY = [np.float32(v) for v in (
    -3.0283174810537603, 6.065830143247177, -5.264110477191794,
    3.2188328371618615, -1.2342631730891853, 0.2668588228746925,
    -0.024825606615893465)]


def _softlog2(z):
    zb = lax.bitcast_convert_type(z, jnp.int32)
    e2 = ((zb >> np.int32(23)) - np.int32(127)).astype(jnp.float32)
    mant = lax.bitcast_convert_type(
        (zb & np.int32(0x7FFFFF)) | np.int32(0x3F800000), jnp.float32)
    acc = jnp.full(mant.shape, _LOG2_POLY[-1], jnp.float32)
    for c in _LOG2_POLY[-2::-1]:
        acc = acc * mant + c
    return e2 + acc


def _tc_stats_body(score_ref, tgt_ref, out_ref):
    i = pl.program_id(0)
    j = pl.program_id(1)

    @pl.when((i == 0) & (j == 0))
    def _init():
        out_ref[...] = jnp.zeros_like(out_ref)

    s = score_ref[0]
    t = tgt_ref[0]
    m = jnp.max(s, axis=0)
    z = jnp.sum(jnp.exp(s - m[None]), axis=0)
    ids = lax.broadcasted_iota(jnp.int32, s.shape, 0)
    st = jnp.sum(jnp.where(ids == t[None], s, np.float32(0.0)), axis=0)
    lse = jnp.log(z) + m
    loss = lse - st
    pg = jnp.exp(st - lse)

    thr = np.float32(_IGNORE_THRESH)
    one = np.float32(1.0)
    zf = np.float32(0.0)
    m_lt = pg < thr
    out_ref[0] = out_ref[0] + jnp.sum(jnp.where(m_lt, one, zf), axis=0)
    out_ref[1] = out_ref[1] + jnp.sum(jnp.where(pg <= thr, one, zf), axis=0)
    out_ref[2] = out_ref[2] + jnp.sum(jnp.where(m_lt, loss, zf), axis=0)


def _tc_stats(score, tgt):
    B, C, H, W = score.shape
    HB = 128
    grid = (B - _B_SC, H // HB)
    return pl.pallas_call(
        _tc_stats_body,
        grid=grid,
        in_specs=[
            pl.BlockSpec((1, C, HB, W),
                         lambda i, j: (i, np.int32(0), j, np.int32(0))),
            pl.BlockSpec((1, HB, W), lambda i, j: (i, j, np.int32(0))),
        ],
        out_specs=pl.BlockSpec((3, W), lambda i, j: (np.int32(0),
                                                     np.int32(0))),
        out_shape=jax.ShapeDtypeStruct((3, W), jnp.float32),
    )(score, tgt)


def _make_sc_dense(B, C, H, W):
    b0 = B - _B_SC
    rows_per_w = (_B_SC * H) // _NW
    w_per_b = H // rows_per_w
    rch = 2
    n_steps = rows_per_w // (2 * rch)
    vecs_per_row = W // _LANES
    mesh = plsc.VectorSubcoreMesh(core_axis_name="c", subcore_axis_name="s")

    @functools.partial(
        pl.kernel,
        out_type=jax.ShapeDtypeStruct((_NW, 3, _LANES), jnp.float32),
        mesh=mesh,
        scratch_types=[
            pltpu.VMEM((C, rch, W), jnp.float32),
            pltpu.VMEM((C, rch, W), jnp.float32),
            pltpu.VMEM((rch, W), jnp.int32),
            pltpu.VMEM((rch, W), jnp.int32),
            pltpu.VMEM((3, _LANES), jnp.float32),
            pltpu.SemaphoreType.DMA,
            pltpu.SemaphoreType.DMA,
        ],
    )
    def sc_dense(score_hbm, tgt_hbm, out_hbm,
                 sb0, sb1, tb0, tb1, acc_v, sem0, sem1):
        wid = lax.axis_index("s") * jnp.int32(_NC) + lax.axis_index("c")
        b = jnp.int32(b0) + wid // jnp.int32(w_per_b)
        r0 = pl.multiple_of((wid % jnp.int32(w_per_b)) * jnp.int32(rows_per_w),
                            8)
        sbufs = (sb0, sb1)
        tbufs = (tb0, tb1)
        sems = (sem0, sem1)

        def issue(slot, r):
            r = pl.multiple_of(r, 2)
            pltpu.async_copy(score_hbm.at[b, :, pl.ds(r, rch), :],
                             sbufs[slot], sems[slot])
            pltpu.async_copy(tgt_hbm.at[b, pl.ds(r, rch), :],
                             tbufs[slot], sems[slot])

        def drain(slot):
            pltpu.make_async_copy(score_hbm.at[b, :, pl.ds(r0, rch), :],
                                  sbufs[slot], sems[slot]).wait()
            pltpu.make_async_copy(tgt_hbm.at[b, pl.ds(r0, rch), :],
                                  tbufs[slot], sems[slot]).wait()

        one = np.float32(1.0)
        zf = np.float32(0.0)
        thr_log = np.float32(0.35667494393873245)

        def compute(sb, tb, carry):
            def row_body(_, inner):
                r = inner[0]

                def col_body(_, cc):
                    off, a_lt, a_le, a_sl = cc
                    offa = pl.multiple_of(off, _LANES)
                    tv = tb[r, pl.ds(offa, _LANES)]
                    z = None
                    st = None
                    for c in range(C):
                        sc_ = sb[c, r, pl.ds(offa, _LANES)]
                        ec = jnp.exp(sc_)
                        hit = jnp.where(tv == np.int32(c), sc_, zf)
                        z = ec if z is None else z + ec
                        st = hit if st is None else st + hit
                    lse = _softlog2(z) * _LN2
                    loss = lse - st
                    m_lt = loss > thr_log
                    a_lt = a_lt + jnp.where(m_lt, one, zf)
                    a_le = a_le + jnp.where(loss >= thr_log, one, zf)
                    a_sl = a_sl + jnp.where(m_lt, loss, zf)
                    return off + np.int32(_LANES), a_lt, a_le, a_sl

                _, a_lt, a_le, a_sl = lax.fori_loop(
                    np.int32(0), np.int32(vecs_per_row), col_body,
                    (np.int32(0),) + inner[1:], unroll=2)
                return (r + np.int32(1), a_lt, a_le, a_sl)

            return lax.fori_loop(np.int32(0), np.int32(rch), row_body,
                                 (np.int32(0),) + carry)[1:]

        issue(0, r0)
        issue(1, r0 + np.int32(rch))

        r_end = r0 + np.int32(rows_per_w)

        def step(_, carry):
            r, a0, a1, a2 = carry
            accs = (a0, a1, a2)
            nxt = r + np.int32(2 * rch)
            more = nxt < r_end
            drain(0)

            @pl.when(more)
            def _():
                issue(0, nxt)

            accs = compute(sb0, tb0, accs)
            drain(1)

            @pl.when(more)
            def _():
                issue(1, nxt + np.int32(rch))

            accs = compute(sb1, tb1, accs)
            return (nxt,) + accs

        zero = jnp.zeros((_LANES,), jnp.float32)
        _, a_lt, a_le, a_sl = lax.fori_loop(
            np.int32(0), np.int32(n_steps), step, (r0, zero, zero, zero))

        acc_v[0] = a_lt
        acc_v[1] = a_le
        acc_v[2] = a_sl
        pltpu.sync_copy(acc_v, out_hbm.at[wid])

    return sc_dense


@functools.lru_cache(maxsize=None)
def _sc_dense_fn(shape):
    return _make_sc_dense(*shape)


def _dense_body(score_ref, tgt_ref, pg_ref, loss_ref):
    s = score_ref[0]
    t = tgt_ref[0]
    m = jnp.max(s, axis=0)
    z = jnp.sum(jnp.exp(s - m[None]), axis=0)
    ids = lax.broadcasted_iota(jnp.int32, s.shape, 0)
    st = jnp.sum(jnp.where(ids == t[None], s, np.float32(0.0)), axis=0)
    lse = jnp.log(z) + m
    loss_ref[0] = lse - st
    pg_ref[0] = jnp.exp(st - lse)


def _dense_stage(score, tgt):
    B, C, H, W = score.shape
    HB = 256
    grid = (B, H // HB)
    return pl.pallas_call(
        _dense_body,
        grid=grid,
        in_specs=[
            pl.BlockSpec((1, C, HB, W),
                         lambda i, j: (i, np.int32(0), j, np.int32(0))),
            pl.BlockSpec((1, HB, W), lambda i, j: (i, j, np.int32(0))),
        ],
        out_specs=[
            pl.BlockSpec((1, HB, W), lambda i, j: (i, j, np.int32(0))),
            pl.BlockSpec((1, HB, W), lambda i, j: (i, j, np.int32(0))),
        ],
        out_shape=[
            jax.ShapeDtypeStruct((B, H, W), jnp.float32),
            jax.ShapeDtypeStruct((B, H, W), jnp.float32),
        ],
    )(score, tgt)


def _make_sc_stats(b_dim, h_dim, w_dim):
    rows_per_w = (b_dim * h_dim) // _NW
    bands = h_dim // rows_per_w
    chr_ = 16
    n_chunks = rows_per_w // chr_
    vecs_per_row = w_dim // _LANES
    mesh = plsc.VectorSubcoreMesh(core_axis_name="c", subcore_axis_name="s")

    @functools.partial(
        pl.kernel,
        out_type=jax.ShapeDtypeStruct((_NW, 3, _LANES), jnp.float32),
        mesh=mesh,
        scratch_types=[
            pltpu.VMEM((chr_, w_dim), jnp.float32),
            pltpu.VMEM((chr_, w_dim), jnp.float32),
            pltpu.VMEM((chr_, w_dim), jnp.float32),
            pltpu.VMEM((chr_, w_dim), jnp.float32),
            pltpu.VMEM((_LANES,), jnp.float32),
            pltpu.VMEM((3, _LANES), jnp.float32),
            pltpu.SemaphoreType.DMA,
            pltpu.SemaphoreType.DMA,
        ],
    )
    def sc_stats(pg_hbm, loss_hbm, thr_hbm, out_hbm,
                 pg_v0, ls_v0, pg_v1, ls_v1, thr_v, acc_v, sem0, sem1):
        wid = lax.axis_index("s") * jnp.int32(_NC) + lax.axis_index("c")
        b = wid // jnp.int32(bands)
        row0 = pl.multiple_of((wid % jnp.int32(bands)) * jnp.int32(rows_per_w),
                              8)
        pltpu.sync_copy(thr_hbm, thr_v)
        thr = thr_v[...]

        pg_bufs = (pg_v0, pg_v1)
        ls_bufs = (ls_v0, ls_v1)
        sems = (sem0, sem1)

        def issue(slot, i):
            r = row0 + np.int32(i * chr_)
            hp = pltpu.async_copy(pg_hbm.at[b, pl.ds(r, chr_), :],
                                  pg_bufs[slot], sems[slot])
            hl = pltpu.async_copy(loss_hbm.at[b, pl.ds(r, chr_), :],
                                  ls_bufs[slot], sems[slot])
            return hp, hl

        handles = [None, None]
        handles[0] = issue(0, 0)
        accs = (jnp.zeros((_LANES,), jnp.float32),
                jnp.zeros((_LANES,), jnp.float32),
                jnp.zeros((_LANES,), jnp.float32))

        one = np.float32(1.0)
        zf = np.float32(0.0)

        for i in range(n_chunks):
            slot = i % 2
            if i + 1 < n_chunks:
                handles[(i + 1) % 2] = issue((i + 1) % 2, i + 1)
            hp, hl = handles[slot]
            hp.wait()
            hl.wait()
            pg_b = pg_bufs[slot]
            ls_b = ls_bufs[slot]

            def row_body(r, carry, pg_b=pg_b, ls_b=ls_b):
                def col_body(_, inner):
                    off, a_lt, a_le, a_sl = inner
                    off_al = pl.multiple_of(off, _LANES)
                    p = pg_b[r, pl.ds(off_al, _LANES)]
                    l = ls_b[r, pl.ds(off_al, _LANES)]
                    m_lt = p < thr
                    a_lt = a_lt + jnp.where(m_lt, one, zf)
                    a_le = a_le + jnp.where(p <= thr, one, zf)
                    a_sl = a_sl + jnp.where(m_lt, l, zf)
                    return off + np.int32(_LANES), a_lt, a_le, a_sl

                _, a_lt, a_le, a_sl = lax.fori_loop(
                    np.int32(0), np.int32(vecs_per_row), col_body,
                    (np.int32(0),) + carry, unroll=8)
                return a_lt, a_le, a_sl

            accs = lax.fori_loop(np.int32(0), np.int32(chr_), row_body, accs)

        acc_v[0] = accs[0]
        acc_v[1] = accs[1]
        acc_v[2] = accs[2]
        pltpu.sync_copy(acc_v, out_hbm.at[wid])

    return sc_stats


@functools.lru_cache(maxsize=None)
def _sc_stats_fn(shape):
    return _make_sc_stats(*shape)


def _sc_stats3(pg, loss, thr):
    thr16 = jnp.full((_LANES,), thr, jnp.float32)
    parts = _sc_stats_fn(pg.shape)(pg, loss, thr16)
    sums = jnp.sum(parts, axis=(0, 2))
    return sums[0], sums[1], sums[2]


def kernel(score, target):
    tgt = lax.bitcast_convert_type(target, jnp.int32)[..., 0]
    sc_parts = _sc_dense_fn(score.shape)(score, tgt)
    tc_parts = _tc_stats(score, tgt)
    c_lt = jnp.sum(tc_parts[0]) + jnp.sum(sc_parts[:, 0, :])
    c_le = jnp.sum(tc_parts[1]) + jnp.sum(sc_parts[:, 1, :])
    s_lt = jnp.sum(tc_parts[2]) + jnp.sum(sc_parts[:, 2, :])
    need = jnp.float32(_MIN_KEPT + 1)

    def common(_):
        return s_lt / jnp.maximum(c_lt, np.float32(1.0))

    def rare(_):
        pg, loss = _dense_stage(score, tgt)

        def cond(lh):
            return lh[0] < lh[1]

        def body(lh):
            lo, hi = lh
            mid = (lo + hi) // jnp.int32(2)
            t = lax.bitcast_convert_type(mid, jnp.float32)
            _, cle_m, _ = _sc_stats3(pg, loss, t)
            ok = cle_m >= need
            return (jnp.where(ok, lo, mid + jnp.int32(1)),
                    jnp.where(ok, mid, hi))

        lo0 = jnp.int32(0)
        hi0 = jnp.int32(0x3F800000)
        lo, _ = lax.while_loop(cond, body, (lo0, hi0))
        vk = lax.bitcast_convert_type(lo, jnp.float32)
        c2, _, s2 = _sc_stats3(pg, loss, vk)
        return s2 / jnp.maximum(c2, np.float32(1.0))

    ohem = lax.cond(c_le >= need, common, rare, None)
    return jnp.float32(_SB_WEIGHTS) * ohem


# speedup vs baseline: 1.6007x; 1.6007x over previous
import functools

import jax
import jax.numpy as jnp
import numpy as np
from jax import lax
from jax.experimental import pallas as pl
from jax.experimental.pallas import tpu as pltpu
from jax.experimental.pallas import tpu_sc as plsc

_IGNORE_THRESH = 0.7
_MIN_KEPT = 100000
_SB_WEIGHTS = 0.5

_NC = 2
_NS = 16
_NW = _NC * _NS
_LANES = 16
_B_SC = 2

_LN2 = np.float32(0.6931471805599453)
_LOG2_POLY = [np.float32(v) for v in (
    -3.0283174810537603, 6.065830143247177, -5.264110477191794,
    3.2188328371618615, -1.2342631730891853, 0.2668588228746925,
    -0.024825606615893465)]


def _softlog2(z):
    zb = lax.bitcast_convert_type(z, jnp.int32)
    e2 = ((zb >> np.int32(23)) - np.int32(127)).astype(jnp.float32)
    mant = lax.bitcast_convert_type(
        (zb & np.int32(0x7FFFFF)) | np.int32(0x3F800000), jnp.float32)
    acc = jnp.full(mant.shape, _LOG2_POLY[-1], jnp.float32)
    for c in _LOG2_POLY[-2::-1]:
        acc = acc * mant + c
    return e2 + acc


def _tc_stats_body(score_ref, tgt_ref, out_ref):
    i = pl.program_id(0)
    j = pl.program_id(1)

    @pl.when((i == 0) & (j == 0))
    def _init():
        out_ref[...] = jnp.zeros_like(out_ref)

    s = score_ref[0]
    t = tgt_ref[0]
    m = jnp.max(s, axis=0)
    z = jnp.sum(jnp.exp(s - m[None]), axis=0)
    ids = lax.broadcasted_iota(jnp.uint32, s.shape, 0)
    st = jnp.sum(jnp.where(ids == t[None], s, np.float32(0.0)), axis=0)
    lse = jnp.log(z) + m
    loss = lse - st
    pg = jnp.exp(st - lse)

    thr = np.float32(_IGNORE_THRESH)
    one = np.float32(1.0)
    zf = np.float32(0.0)
    m_lt = pg < thr
    out_ref[0] = out_ref[0] + jnp.sum(jnp.where(m_lt, one, zf), axis=0)
    out_ref[1] = out_ref[1] + jnp.sum(jnp.where(pg <= thr, one, zf), axis=0)
    out_ref[2] = out_ref[2] + jnp.sum(jnp.where(m_lt, loss, zf), axis=0)


def _tc_stats(score, tgt):
    B, C, H, W = score.shape
    HB = 128
    grid = (B - _B_SC, H // HB)
    return pl.pallas_call(
        _tc_stats_body,
        grid=grid,
        in_specs=[
            pl.BlockSpec((1, C, HB, W),
                         lambda i, j: (i, np.int32(0), j, np.int32(0))),
            pl.BlockSpec((1, HB, W), lambda i, j: (i, j, np.int32(0))),
        ],
        out_specs=pl.BlockSpec((3, W), lambda i, j: (np.int32(0),
                                                     np.int32(0))),
        out_shape=jax.ShapeDtypeStruct((3, W), jnp.float32),
    )(score, tgt)


def _make_sc_dense(B, C, H, W):
    b0 = B - _B_SC
    rows_per_w = (_B_SC * H) // _NW
    w_per_b = H // rows_per_w
    rch = 2
    n_steps = rows_per_w // (2 * rch)
    vecs_per_row = W // _LANES
    mesh = plsc.VectorSubcoreMesh(core_axis_name="c", subcore_axis_name="s")

    @functools.partial(
        pl.kernel,
        out_type=jax.ShapeDtypeStruct((_NW, 3, _LANES), jnp.float32),
        mesh=mesh,
        scratch_types=[
            pltpu.VMEM((C, rch, W), jnp.float32),
            pltpu.VMEM((C, rch, W), jnp.float32),
            pltpu.VMEM((rch, W), jnp.uint32),
            pltpu.VMEM((rch, W), jnp.uint32),
            pltpu.VMEM((3, _LANES), jnp.float32),
            pltpu.SemaphoreType.DMA,
            pltpu.SemaphoreType.DMA,
        ],
    )
    def sc_dense(score_hbm, tgt_hbm, out_hbm,
                 sb0, sb1, tb0, tb1, acc_v, sem0, sem1):
        wid = lax.axis_index("s") * jnp.int32(_NC) + lax.axis_index("c")
        b = jnp.int32(b0) + wid // jnp.int32(w_per_b)
        r0 = pl.multiple_of((wid % jnp.int32(w_per_b)) * jnp.int32(rows_per_w),
                            8)
        sbufs = (sb0, sb1)
        tbufs = (tb0, tb1)
        sems = (sem0, sem1)

        def issue(slot, r):
            r = pl.multiple_of(r, 2)
            pltpu.async_copy(score_hbm.at[b, :, pl.ds(r, rch), :],
                             sbufs[slot], sems[slot])
            pltpu.async_copy(tgt_hbm.at[b, pl.ds(r, rch), :],
                             tbufs[slot], sems[slot])

        def drain(slot):
            pltpu.make_async_copy(score_hbm.at[b, :, pl.ds(r0, rch), :],
                                  sbufs[slot], sems[slot]).wait()
            pltpu.make_async_copy(tgt_hbm.at[b, pl.ds(r0, rch), :],
                                  tbufs[slot], sems[slot]).wait()

        one = np.float32(1.0)
        zf = np.float32(0.0)
        thr_log = np.float32(0.35667494393873245)

        def compute(sb, tb, carry):
            def row_body(_, inner):
                r = inner[0]

                def col_body(_, cc):
                    off, a_lt, a_le, a_sl = cc
                    offa = pl.multiple_of(off, _LANES)
                    tv = tb[r, pl.ds(offa, _LANES)]
                    z = None
                    st = None
                    for c in range(C):
                        sc_ = sb[c, r, pl.ds(offa, _LANES)]
                        ec = jnp.exp(sc_)
                        hit = jnp.where(tv == np.uint32(c), sc_, zf)
                        z = ec if z is None else z + ec
                        st = hit if st is None else st + hit
                    lse = _softlog2(z) * _LN2
                    loss = lse - st
                    m_lt = loss > thr_log
                    a_lt = a_lt + jnp.where(m_lt, one, zf)
                    a_le = a_le + jnp.where(loss >= thr_log, one, zf)
                    a_sl = a_sl + jnp.where(m_lt, loss, zf)
                    return off + np.int32(_LANES), a_lt, a_le, a_sl

                _, a_lt, a_le, a_sl = lax.fori_loop(
                    np.int32(0), np.int32(vecs_per_row), col_body,
                    (np.int32(0),) + inner[1:], unroll=2)
                return (r + np.int32(1), a_lt, a_le, a_sl)

            return lax.fori_loop(np.int32(0), np.int32(rch), row_body,
                                 (np.int32(0),) + carry)[1:]

        issue(0, r0)
        issue(1, r0 + np.int32(rch))

        r_end = r0 + np.int32(rows_per_w)

        def step(_, carry):
            r, a0, a1, a2 = carry
            accs = (a0, a1, a2)
            nxt = r + np.int32(2 * rch)
            more = nxt < r_end
            drain(0)

            @pl.when(more)
            def _():
                issue(0, nxt)

            accs = compute(sb0, tb0, accs)
            drain(1)

            @pl.when(more)
            def _():
                issue(1, nxt + np.int32(rch))

            accs = compute(sb1, tb1, accs)
            return (nxt,) + accs

        zero = jnp.zeros((_LANES,), jnp.float32)
        _, a_lt, a_le, a_sl = lax.fori_loop(
            np.int32(0), np.int32(n_steps), step, (r0, zero, zero, zero))

        acc_v[0] = a_lt
        acc_v[1] = a_le
        acc_v[2] = a_sl
        pltpu.sync_copy(acc_v, out_hbm.at[wid])

    return sc_dense


@functools.lru_cache(maxsize=None)
def _sc_dense_fn(shape):
    return _make_sc_dense(*shape)


def _dense_body(score_ref, tgt_ref, pg_ref, loss_ref):
    s = score_ref[0]
    t = tgt_ref[0]
    m = jnp.max(s, axis=0)
    z = jnp.sum(jnp.exp(s - m[None]), axis=0)
    ids = lax.broadcasted_iota(jnp.uint32, s.shape, 0)
    st = jnp.sum(jnp.where(ids == t[None], s, np.float32(0.0)), axis=0)
    lse = jnp.log(z) + m
    loss_ref[0] = lse - st
    pg_ref[0] = jnp.exp(st - lse)


def _dense_stage(score, tgt):
    B, C, H, W = score.shape
    HB = 256
    grid = (B, H // HB)
    return pl.pallas_call(
        _dense_body,
        grid=grid,
        in_specs=[
            pl.BlockSpec((1, C, HB, W),
                         lambda i, j: (i, np.int32(0), j, np.int32(0))),
            pl.BlockSpec((1, HB, W), lambda i, j: (i, j, np.int32(0))),
        ],
        out_specs=[
            pl.BlockSpec((1, HB, W), lambda i, j: (i, j, np.int32(0))),
            pl.BlockSpec((1, HB, W), lambda i, j: (i, j, np.int32(0))),
        ],
        out_shape=[
            jax.ShapeDtypeStruct((B, H, W), jnp.float32),
            jax.ShapeDtypeStruct((B, H, W), jnp.float32),
        ],
    )(score, tgt)


def _make_sc_stats(b_dim, h_dim, w_dim):
    rows_per_w = (b_dim * h_dim) // _NW
    bands = h_dim // rows_per_w
    chr_ = 16
    n_chunks = rows_per_w // chr_
    vecs_per_row = w_dim // _LANES
    mesh = plsc.VectorSubcoreMesh(core_axis_name="c", subcore_axis_name="s")

    @functools.partial(
        pl.kernel,
        out_type=jax.ShapeDtypeStruct((_NW, 3, _LANES), jnp.float32),
        mesh=mesh,
        scratch_types=[
            pltpu.VMEM((chr_, w_dim), jnp.float32),
            pltpu.VMEM((chr_, w_dim), jnp.float32),
            pltpu.VMEM((chr_, w_dim), jnp.float32),
            pltpu.VMEM((chr_, w_dim), jnp.float32),
            pltpu.VMEM((_LANES,), jnp.float32),
            pltpu.VMEM((3, _LANES), jnp.float32),
            pltpu.SemaphoreType.DMA,
            pltpu.SemaphoreType.DMA,
        ],
    )
    def sc_stats(pg_hbm, loss_hbm, thr_hbm, out_hbm,
                 pg_v0, ls_v0, pg_v1, ls_v1, thr_v, acc_v, sem0, sem1):
        wid = lax.axis_index("s") * jnp.int32(_NC) + lax.axis_index("c")
        b = wid // jnp.int32(bands)
        row0 = pl.multiple_of((wid % jnp.int32(bands)) * jnp.int32(rows_per_w),
                              8)
        pltpu.sync_copy(thr_hbm, thr_v)
        thr = thr_v[...]

        pg_bufs = (pg_v0, pg_v1)
        ls_bufs = (ls_v0, ls_v1)
        sems = (sem0, sem1)

        def issue(slot, i):
            r = row0 + np.int32(i * chr_)
            hp = pltpu.async_copy(pg_hbm.at[b, pl.ds(r, chr_), :],
                                  pg_bufs[slot], sems[slot])
            hl = pltpu.async_copy(loss_hbm.at[b, pl.ds(r, chr_), :],
                                  ls_bufs[slot], sems[slot])
            return hp, hl

        handles = [None, None]
        handles[0] = issue(0, 0)
        accs = (jnp.zeros((_LANES,), jnp.float32),
                jnp.zeros((_LANES,), jnp.float32),
                jnp.zeros((_LANES,), jnp.float32))

        one = np.float32(1.0)
        zf = np.float32(0.0)

        for i in range(n_chunks):
            slot = i % 2
            if i + 1 < n_chunks:
                handles[(i + 1) % 2] = issue((i + 1) % 2, i + 1)
            hp, hl = handles[slot]
            hp.wait()
            hl.wait()
            pg_b = pg_bufs[slot]
            ls_b = ls_bufs[slot]

            def row_body(r, carry, pg_b=pg_b, ls_b=ls_b):
                def col_body(_, inner):
                    off, a_lt, a_le, a_sl = inner
                    off_al = pl.multiple_of(off, _LANES)
                    p = pg_b[r, pl.ds(off_al, _LANES)]
                    l = ls_b[r, pl.ds(off_al, _LANES)]
                    m_lt = p < thr
                    a_lt = a_lt + jnp.where(m_lt, one, zf)
                    a_le = a_le + jnp.where(p <= thr, one, zf)
                    a_sl = a_sl + jnp.where(m_lt, l, zf)
                    return off + np.int32(_LANES), a_lt, a_le, a_sl

                _, a_lt, a_le, a_sl = lax.fori_loop(
                    np.int32(0), np.int32(vecs_per_row), col_body,
                    (np.int32(0),) + carry, unroll=8)
                return a_lt, a_le, a_sl

            accs = lax.fori_loop(np.int32(0), np.int32(chr_), row_body, accs)

        acc_v[0] = accs[0]
        acc_v[1] = accs[1]
        acc_v[2] = accs[2]
        pltpu.sync_copy(acc_v, out_hbm.at[wid])

    return sc_stats


@functools.lru_cache(maxsize=None)
def _sc_stats_fn(shape):
    return _make_sc_stats(*shape)


def _sc_stats3(pg, loss, thr):
    thr16 = jnp.full((_LANES,), thr, jnp.float32)
    parts = _sc_stats_fn(pg.shape)(pg, loss, thr16)
    sums = jnp.sum(parts, axis=(0, 2))
    return sums[0], sums[1], sums[2]


def kernel(score, target):
    tgt = target.astype(jnp.uint32)
    sc_parts = _sc_dense_fn(score.shape)(score, tgt)
    tc_parts = _tc_stats(score, tgt)
    c_lt = jnp.sum(tc_parts[0]) + jnp.sum(sc_parts[:, 0, :])
    c_le = jnp.sum(tc_parts[1]) + jnp.sum(sc_parts[:, 1, :])
    s_lt = jnp.sum(tc_parts[2]) + jnp.sum(sc_parts[:, 2, :])
    need = jnp.float32(_MIN_KEPT + 1)

    def common(_):
        return s_lt / jnp.maximum(c_lt, np.float32(1.0))

    def rare(_):
        pg, loss = _dense_stage(score, tgt)

        def cond(lh):
            return lh[0] < lh[1]

        def body(lh):
            lo, hi = lh
            mid = (lo + hi) // jnp.int32(2)
            t = lax.bitcast_convert_type(mid, jnp.float32)
            _, cle_m, _ = _sc_stats3(pg, loss, t)
            ok = cle_m >= need
            return (jnp.where(ok, lo, mid + jnp.int32(1)),
                    jnp.where(ok, mid, hi))

        lo0 = jnp.int32(0)
        hi0 = jnp.int32(0x3F800000)
        lo, _ = lax.while_loop(cond, body, (lo0, hi0))
        vk = lax.bitcast_convert_type(lo, jnp.float32)
        c2, _, s2 = _sc_stats3(pg, loss, vk)
        return s2 / jnp.maximum(c2, np.float32(1.0))

    ohem = lax.cond(c_le >= need, common, rare, None)
    return jnp.float32(_SB_WEIGHTS) * ohem


# speedup vs baseline: 1.6014x; 1.0005x over previous
import functools

import jax
import jax.numpy as jnp
import numpy as np
from jax import lax
from jax.experimental import pallas as pl
from jax.experimental.pallas import tpu as pltpu
from jax.experimental.pallas import tpu_sc as plsc

_IGNORE_THRESH = 0.7
_MIN_KEPT = 100000
_SB_WEIGHTS = 0.5

_NC = 2
_NS = 16
_NW = _NC * _NS
_LANES = 16
_B_SC = 2

_LN2 = np.float32(0.6931471805599453)
_LOG2_POLY = [np.float32(v) for v in (
    -3.0283174810537603, 6.065830143247177, -5.264110477191794,
    3.2188328371618615, -1.2342631730891853, 0.2668588228746925,
    -0.024825606615893465)]


def _softlog2(z):
    zb = lax.bitcast_convert_type(z, jnp.int32)
    e2 = ((zb >> np.int32(23)) - np.int32(127)).astype(jnp.float32)
    mant = lax.bitcast_convert_type(
        (zb & np.int32(0x7FFFFF)) | np.int32(0x3F800000), jnp.float32)
    acc = jnp.full(mant.shape, _LOG2_POLY[-1], jnp.float32)
    for c in _LOG2_POLY[-2::-1]:
        acc = acc * mant + c
    return e2 + acc


def _tc_stats_body(score_ref, tgt_ref, out_ref):
    i = pl.program_id(0)
    j = pl.program_id(1)

    @pl.when((i == 0) & (j == 0))
    def _init():
        out_ref[...] = jnp.zeros_like(out_ref)

    s = score_ref[0]
    t = tgt_ref[0]
    m = jnp.max(s, axis=0)
    z = jnp.sum(jnp.exp(s - m[None]), axis=0)
    ids = lax.broadcasted_iota(jnp.uint32, s.shape, 0)
    st = jnp.sum(jnp.where(ids == t[None], s, np.float32(0.0)), axis=0)
    lse = jnp.log(z) + m
    loss = lse - st
    pg = jnp.exp(st - lse)

    thr = np.float32(_IGNORE_THRESH)
    one = np.float32(1.0)
    zf = np.float32(0.0)
    m_lt = pg < thr
    out_ref[0] = out_ref[0] + jnp.sum(jnp.where(m_lt, one, zf), axis=0)
    out_ref[1] = out_ref[1] + jnp.sum(jnp.where(pg <= thr, one, zf), axis=0)
    out_ref[2] = out_ref[2] + jnp.sum(jnp.where(m_lt, loss, zf), axis=0)


def _tc_stats(score, tgt):
    B, C, H, W = score.shape
    HB = 128
    grid = (B - _B_SC, H // HB)
    return pl.pallas_call(
        _tc_stats_body,
        grid=grid,
        in_specs=[
            pl.BlockSpec((1, C, HB, W),
                         lambda i, j: (i, np.int32(0), j, np.int32(0))),
            pl.BlockSpec((1, HB, W), lambda i, j: (i, j, np.int32(0))),
        ],
        out_specs=pl.BlockSpec((3, W), lambda i, j: (np.int32(0),
                                                     np.int32(0))),
        out_shape=jax.ShapeDtypeStruct((3, W), jnp.float32),
    )(score, tgt)


def _make_sc_dense(B, C, H, W):
    b0 = B - _B_SC
    rows_per_w = (_B_SC * H) // _NW
    w_per_b = H // rows_per_w
    rch = 2
    n_steps = rows_per_w // (2 * rch)
    vecs_per_row = W // _LANES
    mesh = plsc.VectorSubcoreMesh(core_axis_name="c", subcore_axis_name="s")

    @functools.partial(
        pl.kernel,
        out_type=jax.ShapeDtypeStruct((_NW, 3, _LANES), jnp.float32),
        mesh=mesh,
        scratch_types=[
            pltpu.VMEM((C, rch, W), jnp.float32),
            pltpu.VMEM((C, rch, W), jnp.float32),
            pltpu.VMEM((rch, W), jnp.uint32),
            pltpu.VMEM((rch, W), jnp.uint32),
            pltpu.VMEM((3, _LANES), jnp.float32),
            pltpu.SemaphoreType.DMA,
            pltpu.SemaphoreType.DMA,
        ],
    )
    def sc_dense(score_hbm, tgt_hbm, out_hbm,
                 sb0, sb1, tb0, tb1, acc_v, sem0, sem1):
        wid = lax.axis_index("s") * jnp.int32(_NC) + lax.axis_index("c")
        b = jnp.int32(b0) + wid // jnp.int32(w_per_b)
        r0 = pl.multiple_of((wid % jnp.int32(w_per_b)) * jnp.int32(rows_per_w),
                            8)
        sbufs = (sb0, sb1)
        tbufs = (tb0, tb1)
        sems = (sem0, sem1)

        def issue(slot, r):
            r = pl.multiple_of(r, 2)
            pltpu.async_copy(score_hbm.at[b, :, pl.ds(r, rch), :],
                             sbufs[slot], sems[slot])
            pltpu.async_copy(tgt_hbm.at[b, pl.ds(r, rch), :],
                             tbufs[slot], sems[slot])

        def drain(slot):
            pltpu.make_async_copy(score_hbm.at[b, :, pl.ds(r0, rch), :],
                                  sbufs[slot], sems[slot]).wait()
            pltpu.make_async_copy(tgt_hbm.at[b, pl.ds(r0, rch), :],
                                  tbufs[slot], sems[slot]).wait()

        one = np.float32(1.0)
        zf = np.float32(0.0)
        thr_log = np.float32(0.35667494393873245)

        def compute(sb, tb, carry):
            def row_body(_, inner):
                r = inner[0]

                def col_body(_, cc):
                    off, a_lt, a_le, a_sl = cc
                    offa = pl.multiple_of(off, _LANES)
                    tv = tb[r, pl.ds(offa, _LANES)]
                    z = None
                    st = None
                    for c in range(C):
                        sc_ = sb[c, r, pl.ds(offa, _LANES)]
                        ec = jnp.exp(sc_)
                        hit = jnp.where(tv == np.uint32(c), sc_, zf)
                        z = ec if z is None else z + ec
                        st = hit if st is None else st + hit
                    lse = _softlog2(z) * _LN2
                    loss = lse - st
                    m_lt = loss > thr_log
                    a_lt = a_lt + jnp.where(m_lt, one, zf)
                    a_le = a_le + jnp.where(loss >= thr_log, one, zf)
                    a_sl = a_sl + jnp.where(m_lt, loss, zf)
                    return off + np.int32(_LANES), a_lt, a_le, a_sl

                _, a_lt, a_le, a_sl = lax.fori_loop(
                    np.int32(0), np.int32(vecs_per_row), col_body,
                    (np.int32(0),) + inner[1:], unroll=2)
                return (r + np.int32(1), a_lt, a_le, a_sl)

            return lax.fori_loop(np.int32(0), np.int32(rch), row_body,
                                 (np.int32(0),) + carry)[1:]

        issue(0, r0)
        issue(1, r0 + np.int32(rch))

        r_end = r0 + np.int32(rows_per_w)

        def step(_, carry):
            r, a0, a1, a2 = carry
            accs = (a0, a1, a2)
            nxt = r + np.int32(2 * rch)
            more = nxt < r_end
            drain(0)

            @pl.when(more)
            def _():
                issue(0, nxt)

            accs = compute(sb0, tb0, accs)
            drain(1)

            @pl.when(more)
            def _():
                issue(1, nxt + np.int32(rch))

            accs = compute(sb1, tb1, accs)
            return (nxt,) + accs

        zero = jnp.zeros((_LANES,), jnp.float32)
        _, a_lt, a_le, a_sl = lax.fori_loop(
            np.int32(0), np.int32(n_steps), step, (r0, zero, zero, zero))

        acc_v[0] = a_lt
        acc_v[1] = a_le
        acc_v[2] = a_sl
        pltpu.sync_copy(acc_v, out_hbm.at[wid])

    return sc_dense


@functools.lru_cache(maxsize=None)
def _sc_dense_fn(shape):
    return _make_sc_dense(*shape)


def _dense_body(score_ref, tgt_ref, pg_ref, loss_ref):
    s = score_ref[0]
    t = tgt_ref[0]
    m = jnp.max(s, axis=0)
    z = jnp.sum(jnp.exp(s - m[None]), axis=0)
    ids = lax.broadcasted_iota(jnp.uint32, s.shape, 0)
    st = jnp.sum(jnp.where(ids == t[None], s, np.float32(0.0)), axis=0)
    lse = jnp.log(z) + m
    loss_ref[0] = lse - st
    pg_ref[0] = jnp.exp(st - lse)


def _dense_stage(score, tgt):
    B, C, H, W = score.shape
    HB = 256
    grid = (B, H // HB)
    return pl.pallas_call(
        _dense_body,
        grid=grid,
        in_specs=[
            pl.BlockSpec((1, C, HB, W),
                         lambda i, j: (i, np.int32(0), j, np.int32(0))),
            pl.BlockSpec((1, HB, W), lambda i, j: (i, j, np.int32(0))),
        ],
        out_specs=[
            pl.BlockSpec((1, HB, W), lambda i, j: (i, j, np.int32(0))),
            pl.BlockSpec((1, HB, W), lambda i, j: (i, j, np.int32(0))),
        ],
        out_shape=[
            jax.ShapeDtypeStruct((B, H, W), jnp.float32),
            jax.ShapeDtypeStruct((B, H, W), jnp.float32),
        ],
    )(score, tgt)


def _make_sc_stats(b_dim, h_dim, w_dim):
    rows_per_w = (b_dim * h_dim) // _NW
    bands = h_dim // rows_per_w
    chr_ = 16
    n_chunks = rows_per_w // chr_
    vecs_per_row = w_dim // _LANES
    mesh = plsc.VectorSubcoreMesh(core_axis_name="c", subcore_axis_name="s")

    @functools.partial(
        pl.kernel,
        out_type=jax.ShapeDtypeStruct((_NW, 3, _LANES), jnp.float32),
        mesh=mesh,
        scratch_types=[
            pltpu.VMEM((chr_, w_dim), jnp.float32),
            pltpu.VMEM((chr_, w_dim), jnp.float32),
            pltpu.VMEM((chr_, w_dim), jnp.float32),
            pltpu.VMEM((chr_, w_dim), jnp.float32),
            pltpu.VMEM((_LANES,), jnp.float32),
            pltpu.VMEM((3, _LANES), jnp.float32),
            pltpu.SemaphoreType.DMA,
            pltpu.SemaphoreType.DMA,
        ],
    )
    def sc_stats(pg_hbm, loss_hbm, thr_hbm, out_hbm,
                 pg_v0, ls_v0, pg_v1, ls_v1, thr_v, acc_v, sem0, sem1):
        wid = lax.axis_index("s") * jnp.int32(_NC) + lax.axis_index("c")
        b = wid // jnp.int32(bands)
        row0 = pl.multiple_of((wid % jnp.int32(bands)) * jnp.int32(rows_per_w),
                              8)
        pltpu.sync_copy(thr_hbm, thr_v)
        thr = thr_v[...]

        pg_bufs = (pg_v0, pg_v1)
        ls_bufs = (ls_v0, ls_v1)
        sems = (sem0, sem1)

        def issue(slot, i):
            r = row0 + np.int32(i * chr_)
            hp = pltpu.async_copy(pg_hbm.at[b, pl.ds(r, chr_), :],
                                  pg_bufs[slot], sems[slot])
            hl = pltpu.async_copy(loss_hbm.at[b, pl.ds(r, chr_), :],
                                  ls_bufs[slot], sems[slot])
            return hp, hl

        handles = [None, None]
        handles[0] = issue(0, 0)
        accs = (jnp.zeros((_LANES,), jnp.float32),
                jnp.zeros((_LANES,), jnp.float32),
                jnp.zeros((_LANES,), jnp.float32))

        one = np.float32(1.0)
        zf = np.float32(0.0)

        for i in range(n_chunks):
            slot = i % 2
            if i + 1 < n_chunks:
                handles[(i + 1) % 2] = issue((i + 1) % 2, i + 1)
            hp, hl = handles[slot]
            hp.wait()
            hl.wait()
            pg_b = pg_bufs[slot]
            ls_b = ls_bufs[slot]

            def row_body(r, carry, pg_b=pg_b, ls_b=ls_b):
                def col_body(_, inner):
                    off, a_lt, a_le, a_sl = inner
                    off_al = pl.multiple_of(off, _LANES)
                    p = pg_b[r, pl.ds(off_al, _LANES)]
                    l = ls_b[r, pl.ds(off_al, _LANES)]
                    m_lt = p < thr
                    a_lt = a_lt + jnp.where(m_lt, one, zf)
                    a_le = a_le + jnp.where(p <= thr, one, zf)
                    a_sl = a_sl + jnp.where(m_lt, l, zf)
                    return off + np.int32(_LANES), a_lt, a_le, a_sl

                _, a_lt, a_le, a_sl = lax.fori_loop(
                    np.int32(0), np.int32(vecs_per_row), col_body,
                    (np.int32(0),) + carry, unroll=8)
                return a_lt, a_le, a_sl

            accs = lax.fori_loop(np.int32(0), np.int32(chr_), row_body, accs)

        acc_v[0] = accs[0]
        acc_v[1] = accs[1]
        acc_v[2] = accs[2]
        pltpu.sync_copy(acc_v, out_hbm.at[wid])

    return sc_stats


@functools.lru_cache(maxsize=None)
def _sc_stats_fn(shape):
    return _make_sc_stats(*shape)


def _sc_stats3(pg, loss, thr):
    thr16 = jnp.full((_LANES,), thr, jnp.float32)
    parts = _sc_stats_fn(pg.shape)(pg, loss, thr16)
    sums = jnp.sum(parts, axis=(0, 2))
    return sums[0], sums[1], sums[2]


def kernel(score, target):
    B, _, H, W = score.shape
    tgt = target.reshape(-1).astype(jnp.uint32).reshape(B, H, W)
    sc_parts = _sc_dense_fn(score.shape)(score, tgt)
    tc_parts = _tc_stats(score, tgt)
    c_lt = jnp.sum(tc_parts[0]) + jnp.sum(sc_parts[:, 0, :])
    c_le = jnp.sum(tc_parts[1]) + jnp.sum(sc_parts[:, 1, :])
    s_lt = jnp.sum(tc_parts[2]) + jnp.sum(sc_parts[:, 2, :])
    need = jnp.float32(_MIN_KEPT + 1)

    def common(_):
        return s_lt / jnp.maximum(c_lt, np.float32(1.0))

    def rare(_):
        pg, loss = _dense_stage(score, tgt)

        def cond(lh):
            return lh[0] < lh[1]

        def body(lh):
            lo, hi = lh
            mid = (lo + hi) // jnp.int32(2)
            t = lax.bitcast_convert_type(mid, jnp.float32)
            _, cle_m, _ = _sc_stats3(pg, loss, t)
            ok = cle_m >= need
            return (jnp.where(ok, lo, mid + jnp.int32(1)),
                    jnp.where(ok, mid, hi))

        lo0 = jnp.int32(0)
        hi0 = jnp.int32(0x3F800000)
        lo, _ = lax.while_loop(cond, body, (lo0, hi0))
        vk = lax.bitcast_convert_type(lo, jnp.float32)
        c2, _, s2 = _sc_stats3(pg, loss, vk)
        return s2 / jnp.maximum(c2, np.float32(1.0))

    ohem = lax.cond(c_le >= need, common, rare, None)
    return jnp.float32(_SB_WEIGHTS) * ohem
